# Initial kernel scaffold; baseline (speedup 1.0000x reference)
import jax
import jax.numpy as jnp
from jax.experimental import pallas as pl


def kernel(filtered_sino, rows, cols, vals):
    raise NotImplementedError("write your pallas kernel here")


# speedup vs baseline: 29.8829x; 29.8829x over previous
import functools

import jax
import jax.numpy as jnp
from jax import lax
from jax.experimental import pallas as pl
from jax.experimental.pallas import tpu as pltpu
from jax.experimental.pallas import tpu_sc as plsc

PROJ = 1000
DET = 513
IMG = 362
BSZ = 5
NROWS = IMG * IMG
NCOLS = DET * PROJ

NC = 2
NS = 16
NW = NC * NS
RPW = 4096
BANK = RPW * 8
K = 2048
GSUB = 128


def _sc_body(rows_hbm, cols_hbm, vals_hbm, table_hbm, blo_hbm, bhi_hbm,
             out_hbm, bounds_v, rows_v, cols_v, vals_v, gath_v, acc_v, sem):
    wid = lax.axis_index("s") * NC + lax.axis_index("c")
    iota = lax.iota(jnp.int32, 16)
    lane8 = jnp.bitwise_and(iota, 7)
    half = jnp.right_shift(iota, 3)
    laneoff = lane8 + half * BANK
    lanelt6 = lane8 < 6
    rowlo = wid * RPW
    rowhi = rowlo + RPW

    zero16 = jnp.zeros((16,), jnp.float32)

    def zbody(i, c):
        acc_v[pl.ds(i * 16, 16)] = zero16
        return c

    lax.fori_loop(0, 2 * BANK // 16, zbody, 0)

    pltpu.sync_copy(blo_hbm, bounds_v.at[pl.ds(0, NW)])
    pltpu.sync_copy(bhi_hbm, bounds_v.at[pl.ds(NW, NW)])

    myblo = bounds_v[pl.ds(wid, 16)][0]
    mybhi = bounds_v[pl.ds(NW + wid, 16)][0]

    def block_body(g, c):
        base = pl.multiple_of(g * K, 8)
        pltpu.sync_copy(rows_hbm.at[pl.ds(base, K)], rows_v)
        pltpu.sync_copy(cols_hbm.at[pl.ds(base, K)], cols_v)
        pltpu.sync_copy(vals_hbm.at[pl.ds(base, K)], vals_v)
        copies = []
        for j in range(K // GSUB):
            copies.append(pltpu.async_copy(
                table_hbm.at[cols_v.at[pl.ds(j * GSUB, GSUB)]],
                gath_v.at[pl.ds(j * GSUB, GSUB)],
                sem))
        for cp in copies:
            cp.wait()

        def pair_body(p, cc):
            eidx = half + p * 2
            rows_e = plsc.load_gather(rows_v, [eidx])
            vals_e = plsc.load_gather(vals_v, [eidx])
            g16 = plsc.load_gather(gath_v, [eidx, lane8])
            prod = g16 * vals_e
            sidx = (rows_e - rowlo) * 8 + laneoff
            m = jnp.logical_and(
                jnp.logical_and(rows_e >= rowlo, rows_e < rowhi), lanelt6)
            plsc.addupdate_scatter(acc_v, [sidx], prod, mask=m)
            return cc

        lax.fori_loop(0, K // 2, pair_body, 0)
        return c

    lax.fori_loop(myblo, mybhi, block_body, 0)

    sensoff = half * 8 + 5

    def merge_body(p, c):
        b16 = p * 16
        acc_v[pl.ds(b16, 16)] = acc_v[pl.ds(b16, 16)] + acc_v[pl.ds(BANK + b16, 16)]
        return c

    lax.fori_loop(0, BANK // 16, merge_body, 0)

    def div_body(p, c):
        b16 = p * 16
        v = acc_v[pl.ds(b16, 16)]
        s = plsc.load_gather(acc_v, [b16 + sensoff])
        acc_v[pl.ds(b16, 16)] = v / s
        return c

    lax.fori_loop(0, BANK // 16, div_body, 0)

    pltpu.sync_copy(acc_v.at[pl.ds(0, BANK)],
                    out_hbm.at[pl.ds(wid * BANK, BANK)])


_sc_call = functools.partial(
    pl.kernel,
    out_type=jax.ShapeDtypeStruct((NW * BANK,), jnp.float32),
    mesh=plsc.VectorSubcoreMesh(
        core_axis_name="c", subcore_axis_name="s",
        num_cores=NC, num_subcores=NS),
    scratch_types=[
        pltpu.VMEM((2 * NW + 32,), jnp.int32),
        pltpu.VMEM((K,), jnp.int32),
        pltpu.VMEM((K,), jnp.int32),
        pltpu.VMEM((K,), jnp.float32),
        pltpu.VMEM((K, 8), jnp.float32),
        pltpu.VMEM((2 * BANK,), jnp.float32),
        pltpu.SemaphoreType.DMA,
    ],
    compiler_params=pltpu.CompilerParams(
        needs_layout_passes=False, use_tc_tiling_on_sc=False),
)(_sc_body)


def kernel(filtered_sino, rows, cols, vals):
    flat = filtered_sino.reshape(BSZ, NCOLS)
    table = jnp.concatenate(
        [flat.T,
         jnp.ones((NCOLS, 1), jnp.float32),
         jnp.zeros((NCOLS, 2), jnp.float32)], axis=1)

    nnz = rows.shape[0]
    nnz_pad = ((nnz + K - 1) // K) * K
    pad = nnz_pad - nnz
    rows_p = jnp.concatenate(
        [rows, jnp.full((pad,), NROWS - 1, jnp.int32)])
    cols_p = jnp.concatenate([cols, jnp.zeros((pad,), jnp.int32)])
    vals_p = jnp.concatenate([vals, jnp.zeros((pad,), jnp.float32)])

    bound_rows = jnp.arange(NW + 1, dtype=jnp.int32) * RPW
    cut = jnp.searchsorted(rows_p, bound_rows, side="left").astype(jnp.int32)
    blo = cut[:NW] // K
    bhi = (cut[1:] + K - 1) // K

    out = _sc_call(rows_p, cols_p, vals_p, table, blo, bhi)
    res = out.reshape(NW * RPW, 8)[:NROWS, :BSZ]
    return res.T.reshape(BSZ, IMG, IMG, 1)
